# baseline (speedup 1.0000x reference)
import jax
import jax.numpy as jnp
from jax.experimental import pallas as pl
from jax.experimental.pallas import tpu as pltpu

N = 15
NP = 16
D = 512
H = 3
L = 3
NEG = -1e30


def _layer_kernel(x0_ref, news_ref,
                  ws00, ws01, ws02, ws10, ws11, ws12, ws20, ws21, ws22,
                  wd10, wd11, wd12, wd20, wd21, wd22,
                  as_ref, ad_ref, b_ref, out_ref):
    i = pl.program_id(0)

    x = jnp.where(i == 0, x0_ref[...], out_ref[...])

    w_avg = (ws00[0, 0] + ws01[0, 0] + ws02[0, 0]) * (1.0 / 3.0)
    acc = jnp.dot(news_ref[0], w_avg, preferred_element_type=jnp.float32)

    src_ok = jax.lax.broadcasted_iota(jnp.int32, (NP, NP), 0) < N

    for r, ws_heads, wd_heads in (
            (1, (ws10, ws11, ws12), (wd10, wd11, wd12)),
            (2, (ws20, ws21, ws22), (wd20, wd21, wd22))):
        a_s = as_ref[0, r]
        a_d = ad_ref[0, r]
        for h in range(H):
            hs_h = jnp.dot(x, ws_heads[h][0, 0],
                           preferred_element_type=jnp.float32)
            hd_h = jnp.dot(x, wd_heads[h][0, 0],
                           preferred_element_type=jnp.float32)
            al_s = jnp.sum(hs_h * a_s[h][None, :], axis=1, keepdims=True)
            al_d = jnp.sum(hd_h * a_d[h][None, :], axis=1, keepdims=True)
            alpha = al_s + jnp.transpose(al_d)
            alpha = jnp.where(alpha > 0, alpha, 0.2 * alpha)
            alpha = jnp.where(src_ok, alpha, NEG)
            amax = jnp.max(alpha, axis=0, keepdims=True)
            e = jnp.exp(alpha - amax)
            denom = jnp.sum(e, axis=0, keepdims=True)
            att = e / (denom + 1e-16)
            acc = acc + (1.0 / H) * jax.lax.dot_general(
                att, hs_h, (((0,), (0,)), ((), ())),
                preferred_element_type=jnp.float32)

    b = b_ref[0]
    acc = acc + (b[0] + b[1] + b[2])[None, :]
    out_ref[...] = jax.nn.sigmoid(acc * (1.0 / 3.0))


@jax.jit
def kernel(company_features, daily_news_features, W_src, W_dst, att_src,
           att_dst, bias):
    x0 = jnp.zeros((NP, D), jnp.float32).at[:N].set(company_features)
    news = jnp.zeros((L, NP, D), jnp.float32).at[:, :N].set(
        daily_news_features)

    def wspec(r, h):
        return pl.BlockSpec((1, 1, D, D),
                            lambda i, _r=r, _h=h: (i, _r, 0, _h))

    ws_specs = [wspec(r, h) for r in range(3) for h in range(H)]
    wd_specs = [wspec(r, h) for r in (1, 2) for h in range(H)]

    out = pl.pallas_call(
        _layer_kernel,
        grid=(L,),
        in_specs=[
            pl.BlockSpec((NP, D), lambda i: (0, 0)),
            pl.BlockSpec((1, NP, D), lambda i: (i, 0, 0)),
            *ws_specs,
            *wd_specs,
            pl.BlockSpec((1, 3, H, D), lambda i: (i, 0, 0, 0)),
            pl.BlockSpec((1, 3, H, D), lambda i: (i, 0, 0, 0)),
            pl.BlockSpec((1, 3, D), lambda i: (i, 0, 0)),
        ],
        out_specs=pl.BlockSpec((NP, D), lambda i: (0, 0)),
        out_shape=jax.ShapeDtypeStruct((NP, D), jnp.float32),
        compiler_params=pltpu.CompilerParams(
            dimension_semantics=("arbitrary",)),
    )(x0, news, *([W_src] * 9), *([W_dst] * 6), att_src, att_dst, bias)
    return out[:N]
